# baseline (speedup 1.0000x reference)
import jax
import jax.numpy as jnp
from jax import lax
from jax.experimental import pallas as pl
from jax.experimental.pallas import tpu as pltpu
from jax.experimental.pallas import tpu_sc as plsc

NU = 25000
NN = 50000
NP = 50048
HALF = 32
E0 = 800000
BATCH = 4096
NC, NS = 2, 16
CH = 384
NCHUNK = 132
PT = NCHUNK * CH
EPAD = PT * NS
ZROWS = NP // NS
ZFULL = ZROWS // CH
ZREM = ZROWS % CH
PB = BATCH // NS


def _body(x0_h, epk_h, pidx_h, fw_h,
          gam_h, h1_h, h2_h, h3_h,
          eb0, eb1, eb2, eb3, rows0, rows1, rowsF,
          fw_v, acc, semA, semB, semI0, semI1, semI2, semI3, semZ):
  c = lax.axis_index("c")
  s = lax.axis_index("s")

  ebs = [eb0, eb1, eb2, eb3]
  semIs = [semI0, semI1, semI2, semI3]
  zeros16 = jnp.zeros((16,), jnp.float32)

  def pf_idx(g, k):
    gg = jnp.where(g < NCHUNK, g, 0)
    pltpu.async_copy(epk_h.at[c, s * NCHUNK + gg], ebs[k], semIs[k])

  def wt_idx(k):
    pltpu.make_async_copy(epk_h.at[c, 0], ebs[k], semIs[k]).wait()

  def fire_gather(k, rowsb, sem, tab):
    pltpu.async_copy(tab.at[ebs[k].at[0]], rowsb, sem)

  def wt_gather(k, rowsb, sem, tab):
    pltpu.make_async_copy(tab.at[ebs[k].at[0]], rowsb, sem).wait()

  def scale(k, rowsb):
    eb = ebs[k]

    @pl.loop(0, CH // 16)
    def _scale(gg):
      wv = plsc.bitcast(eb[2, pl.ds(gg * 16, 16)], jnp.float32)
      for i in range(16):
        e = gg * 16 + i
        w = wv[i]
        a, b = plsc.unpack(rowsb[e], format=plsc.PackFormat.INTERLEAVED)
        rowsF[e, pl.ds(0, 16)] = a * w
        rowsF[e, pl.ds(16, 16)] = b * w

  def scatter(k):
    pltpu.sync_copy(rowsF, acc.at[ebs[k].at[1]], add=True)

  hin = [x0_h, h1_h, h2_h]
  hout = [h1_h, h2_h, h3_h]

  for k in range(4):
    pf_idx(jnp.int32(k), k)

  zb = s * ZROWS

  for l in range(3):
    @pl.loop(0, CH)
    def _zero(e):
      rowsF[e, pl.ds(0, 16)] = zeros16
      rowsF[e, pl.ds(16, 16)] = zeros16

    @pl.loop(0, ZFULL)
    def _zf(k):
      pltpu.async_copy(rowsF, acc.at[pl.ds(zb + k * CH, CH)], semZ)

    pltpu.async_copy(rowsF.at[pl.ds(0, ZREM)],
                     acc.at[pl.ds(zb + ZFULL * CH, ZREM)], semZ)

    @pl.loop(0, ZFULL)
    def _zw(k):
      pltpu.make_async_copy(rowsF, acc.at[pl.ds(zb + k * CH, CH)],
                            semZ).wait()

    pltpu.make_async_copy(rowsF.at[pl.ds(0, ZREM)],
                          acc.at[pl.ds(zb + ZFULL * CH, ZREM)], semZ).wait()
    plsc.subcore_barrier()

    tab = hin[l]
    wt_idx(0)
    fire_gather(0, rows0, semA, tab)
    wt_idx(1)
    fire_gather(1, rows1, semB, tab)

    @pl.loop(0, NCHUNK // 4)
    def _edges(u):
      g = 4 * u
      wt_gather(0, rows0, semA, tab)
      scale(0, rows0)
      scatter(0)
      wt_idx(2)
      fire_gather(2, rows0, semA, tab)
      pf_idx(g + 4, 0)
      wt_gather(1, rows1, semB, tab)
      scale(1, rows1)
      scatter(1)
      wt_idx(3)
      fire_gather(3, rows1, semB, tab)
      pf_idx(g + 5, 1)
      wt_gather(2, rows0, semA, tab)
      scale(2, rows0)
      scatter(2)
      wt_idx(0)
      fire_gather(0, rows0, semA, tab)
      pf_idx(g + 6, 2)
      wt_gather(3, rows1, semB, tab)
      scale(3, rows1)
      scatter(3)
      wt_idx(1)
      fire_gather(1, rows1, semB, tab)
      pf_idx(g + 7, 3)

    wt_gather(0, rows0, semA, tab)
    wt_gather(1, rows1, semB, tab)
    wt_idx(2)
    wt_idx(3)
    plsc.subcore_barrier()

    for k in range(4):
      pf_idx(jnp.int32(k), k)

    ob = c * NP + zb
    hl = hout[l]

    @pl.loop(0, ZFULL)
    def _wb(k, hl=hl):
      pltpu.sync_copy(acc.at[pl.ds(zb + k * CH, CH)], rowsF)

      @pl.loop(0, CH)
      def _pack(e):
        rows0[e] = plsc.pack(rowsF[e, pl.ds(0, 16)],
                             rowsF[e, pl.ds(16, 16)],
                             format=plsc.PackFormat.INTERLEAVED)

      pltpu.sync_copy(rows0, hl.at[pl.ds(ob + k * CH, CH)])

    pltpu.sync_copy(acc.at[pl.ds(zb + ZFULL * CH, ZREM)],
                    rowsF.at[pl.ds(0, ZREM)])

    @pl.loop(0, ZREM)
    def _packr(e):
      rows0[e] = plsc.pack(rowsF[e, pl.ds(0, 16)],
                           rowsF[e, pl.ds(16, 16)],
                           format=plsc.PackFormat.INTERLEAVED)

    pltpu.sync_copy(rows0.at[pl.ds(0, ZREM)],
                    hl.at[pl.ds(ob + ZFULL * CH, ZREM)])

  for k in range(4):
    wt_idx(k)

  pltpu.sync_copy(fw_h, fw_v)
  pltpu.sync_copy(pidx_h.at[c, pl.ds(2 * s, 2)], eb0.at[pl.ds(0, 2)])

  hs = [x0_h, h1_h, h2_h, h3_h]
  steps = ([(0, l, j) for l in range(4) for j in range(2)]
           + [(1, l, j) for l in range(4) for j in range(2)])

  def _stage(n):
    return (rows0, semA) if n % 2 == 0 else (rows1, semB)

  def _fire_final(n):
    half, l, j = steps[n]
    stg, sem = _stage(n)
    pltpu.async_copy(hs[l].at[eb0.at[half, pl.ds(j * 128, 128)]],
                     stg.at[pl.ds(0, 128)], sem)

  def _wait_final(n):
    half, l, j = steps[n]
    stg, sem = _stage(n)
    pltpu.make_async_copy(hs[l].at[eb0.at[half, pl.ds(j * 128, 128)]],
                          stg.at[pl.ds(0, 128)], sem).wait()

  @pl.loop(0, 8)
  def _zg(r):
    rowsF[256 + r, pl.ds(0, 16)] = zeros16
    rowsF[256 + r, pl.ds(16, 16)] = zeros16

  iota = lax.iota(jnp.int32, 16)
  _fire_final(0)
  for n in range(16):
    _wait_final(n)
    if n + 1 < 16:
      _fire_final(n + 1)
    half, l, j = steps[n]
    fwl = fw_v[l, pl.ds(0, 16)]
    stg, _ = _stage(n)

    if half == 0:
      @pl.loop(0, 128)
      def _fuse(e, l=l, j=j, fwl=fwl, stg=stg):
        a, b = plsc.unpack(stg[e], format=plsc.PackFormat.INTERLEAVED)
        row = j * 128 + e
        if l == 0:
          rowsF[row, pl.ds(0, 16)] = fwl * a
          rowsF[row, pl.ds(16, 16)] = fwl * b
        else:
          rowsF[row, pl.ds(0, 16)] = rowsF[row, pl.ds(0, 16)] + fwl * a
          rowsF[row, pl.ds(16, 16)] = rowsF[row, pl.ds(16, 16)] + fwl * b
    else:
      @pl.loop(0, 8)
      def _dot(grp, l=l, j=j, fwl=fwl, stg=stg):
        e0 = j * 128 + grp * 16
        accv = jnp.zeros((16,), jnp.float32)
        for i in range(16):
          e = e0 + i
          a, b = plsc.unpack(stg[grp * 16 + i],
                             format=plsc.PackFormat.INTERLEAVED)
          v = (rowsF[e, pl.ds(0, 16)] * a
               + rowsF[e, pl.ds(16, 16)] * b)
          accv = accv + jnp.where(iota == i, jnp.sum(v), 0.0)
        grow = 256 + e0 // 32
        gcol = (e0 % 32) // 16 * 16
        rowsF[grow, pl.ds(gcol, 16)] = (rowsF[grow, pl.ds(gcol, 16)]
                                        + fwl * accv)

  pltpu.sync_copy(rowsF.at[pl.ds(256, 8)],
                  gam_h.at[pl.ds((c * NS + s) * 8, 8)])


@jax.jit
def _run(x0, epk, pidx, fwv):
  mesh = plsc.VectorSubcoreMesh(core_axis_name="c", subcore_axis_name="s",
                                num_cores=NC, num_subcores=NS)
  f = pl.kernel(
      _body,
      out_type=[
          jax.ShapeDtypeStruct((NC * NS * 8, HALF), jnp.float32),
          jax.ShapeDtypeStruct((NC * NP, HALF), jnp.bfloat16),
          jax.ShapeDtypeStruct((NC * NP, HALF), jnp.bfloat16),
          jax.ShapeDtypeStruct((NC * NP, HALF), jnp.bfloat16),
      ],
      mesh=mesh,
      compiler_params=pltpu.CompilerParams(use_tc_tiling_on_sc=False,
                                           needs_layout_passes=False),
      scratch_types=[
          pltpu.VMEM((3, CH), jnp.int32),
          pltpu.VMEM((3, CH), jnp.int32),
          pltpu.VMEM((3, CH), jnp.int32),
          pltpu.VMEM((3, CH), jnp.int32),
          pltpu.VMEM((CH, HALF), jnp.bfloat16),
          pltpu.VMEM((CH, HALF), jnp.bfloat16),
          pltpu.VMEM((CH, HALF), jnp.float32),
          pltpu.VMEM((8, 16), jnp.float32),
          pltpu.VMEM_SHARED((NP, HALF), jnp.float32),
          pltpu.SemaphoreType.DMA,
          pltpu.SemaphoreType.DMA,
          pltpu.SemaphoreType.DMA,
          pltpu.SemaphoreType.DMA,
          pltpu.SemaphoreType.DMA,
          pltpu.SemaphoreType.DMA,
          pltpu.SemaphoreType.DMA,
      ],
  )
  return f(x0, epk, pidx, fwv)


def kernel(users, items, user_emb, item_emb, edge_src, edge_dst, edge_w,
           fw1, fw2, fw3, fw4):
  all_emb = jnp.concatenate([user_emb, item_emb], axis=0)
  npad = jnp.zeros((NP - NN, HALF), jnp.float32)
  x0 = jnp.concatenate(
      [all_emb[:, :HALF], npad, all_emb[:, HALF:], npad],
      0).astype(jnp.bfloat16)

  pad = EPAD - E0
  esrc = jnp.concatenate([edge_src.astype(jnp.int32),
                          jnp.zeros((pad,), jnp.int32)]).reshape(
                              NS * NCHUNK, CH)
  edst = jnp.concatenate([edge_dst.astype(jnp.int32),
                          jnp.zeros((pad,), jnp.int32)]).reshape(
                              NS * NCHUNK, CH)
  wbits = lax.bitcast_convert_type(
      jnp.concatenate([edge_w.astype(jnp.float32),
                       jnp.zeros((pad,), jnp.float32)]),
      jnp.int32).reshape(NS * NCHUNK, CH)
  epk = jnp.stack([
      jnp.stack([esrc, edst, wbits], axis=1),
      jnp.stack([esrc + NP, edst, wbits], axis=1),
  ], axis=0)

  zpad = jnp.zeros((NS, CH - PB), jnp.int32)
  u2 = jnp.concatenate([users.astype(jnp.int32).reshape(NS, PB), zpad], 1)
  i2 = jnp.concatenate([(items.astype(jnp.int32) + NU).reshape(NS, PB),
                        zpad], 1)
  pidx0 = jnp.stack([u2, i2], axis=1).reshape(NS * 2, CH)
  pidx = jnp.stack([pidx0, pidx0 + NP], axis=0)

  fwv = jnp.zeros((8, 16), jnp.float32)
  fwv = fwv.at[0:4].set(
      jnp.broadcast_to(
          jnp.stack([fw1, fw2, fw3, fw4]).reshape(4, 1).astype(jnp.float32),
          (4, 16)))

  gam, _, _, _ = _run(x0, epk, pidx, fwv)
  gam = gam.reshape(NC, BATCH)
  return gam[0] + gam[1]


# speedup vs baseline: 1.2427x; 1.2427x over previous
import jax
import jax.numpy as jnp
from jax import lax
from jax.experimental import pallas as pl
from jax.experimental.pallas import tpu as pltpu
from jax.experimental.pallas import tpu_sc as plsc

NU = 25000
NN = 50000
NP = 50048
HALF = 32
E0 = 800000
BATCH = 4096
NC, NS = 2, 16
CH = 384
NCHUNK = 132
PT = NCHUNK * CH
EPAD = PT * NS
ZROWS = NP // NS
ZFULL = ZROWS // CH
ZREM = ZROWS % CH
PB = BATCH // NS


def _body(x0_h, epk_h, pidx_h, fw_h,
          gam_h, h1_h, h2_h, h3_h,
          eb0, eb1, eb2, eb3, rows0, rows1,
          fw_v, acc, semA, semB, semI0, semI1, semI2, semI3, semZ):
  c = lax.axis_index("c")
  s = lax.axis_index("s")

  ebs = [eb0, eb1, eb2, eb3]
  semIs = [semI0, semI1, semI2, semI3]
  zeros16 = jnp.zeros((16,), jnp.float32)

  def pf_idx(g, k):
    gg = jnp.where(g < NCHUNK, g, 0)
    pltpu.async_copy(epk_h.at[c, s * NCHUNK + gg], ebs[k], semIs[k])

  def wt_idx(k):
    pltpu.make_async_copy(epk_h.at[c, 0], ebs[k], semIs[k]).wait()

  def fire_gather(k, rowsb, sem, tab):
    pltpu.async_copy(tab.at[ebs[k].at[0]], rowsb, sem)

  def wt_gather(k, rowsb, sem, tab):
    pltpu.make_async_copy(tab.at[ebs[k].at[0]], rowsb, sem).wait()

  def scale(k, rowsb):
    eb = ebs[k]

    @pl.loop(0, CH // 16)
    def _scale(gg):
      wv = plsc.bitcast(eb[2, pl.ds(gg * 16, 16)], jnp.float32)
      for i in range(16):
        e = gg * 16 + i
        w = wv[i]
        rowsb[e, pl.ds(0, 16)] = rowsb[e, pl.ds(0, 16)] * w
        rowsb[e, pl.ds(16, 16)] = rowsb[e, pl.ds(16, 16)] * w

  def scatter(k, rowsb):
    pltpu.sync_copy(rowsb, acc.at[ebs[k].at[1]], add=True)

  hin = [x0_h, h1_h, h2_h]
  hout = [h1_h, h2_h, h3_h]

  zb = s * ZROWS
  ob0 = zb

  for k in range(4):
    pf_idx(jnp.int32(k), k)

  for l in range(3):
    @pl.loop(0, CH)
    def _zero(e):
      rows0[e, pl.ds(0, 16)] = zeros16
      rows0[e, pl.ds(16, 16)] = zeros16

    for k in range(ZFULL):
      pltpu.async_copy(rows0, acc.at[pl.ds(zb + k * CH, CH)], semZ)
    pltpu.async_copy(rows0.at[pl.ds(0, ZREM)],
                     acc.at[pl.ds(zb + ZFULL * CH, ZREM)], semZ)
    for k in range(ZFULL):
      pltpu.make_async_copy(rows0, acc.at[pl.ds(zb + k * CH, CH)],
                            semZ).wait()
    pltpu.make_async_copy(rows0.at[pl.ds(0, ZREM)],
                          acc.at[pl.ds(zb + ZFULL * CH, ZREM)], semZ).wait()
    plsc.subcore_barrier()

    tab = hin[l]
    wt_idx(0)
    fire_gather(0, rows0, semA, tab)
    wt_idx(1)
    fire_gather(1, rows1, semB, tab)

    @pl.loop(0, NCHUNK // 4)
    def _edges(u):
      g = 4 * u
      wt_gather(0, rows0, semA, tab)
      scale(0, rows0)
      scatter(0, rows0)
      wt_idx(2)
      fire_gather(2, rows0, semA, tab)
      pf_idx(g + 4, 0)
      wt_gather(1, rows1, semB, tab)
      scale(1, rows1)
      scatter(1, rows1)
      wt_idx(3)
      fire_gather(3, rows1, semB, tab)
      pf_idx(g + 5, 1)
      wt_gather(2, rows0, semA, tab)
      scale(2, rows0)
      scatter(2, rows0)
      wt_idx(0)
      fire_gather(0, rows0, semA, tab)
      pf_idx(g + 6, 2)
      wt_gather(3, rows1, semB, tab)
      scale(3, rows1)
      scatter(3, rows1)
      wt_idx(1)
      fire_gather(1, rows1, semB, tab)
      pf_idx(g + 7, 3)

    wt_gather(0, rows0, semA, tab)
    wt_gather(1, rows1, semB, tab)
    wt_idx(2)
    wt_idx(3)
    plsc.subcore_barrier()

    for k in range(4):
      pf_idx(jnp.int32(k), k)

    ob = c * NP + zb
    for k in range(ZFULL):
      pltpu.async_copy(acc.at[pl.ds(zb + k * CH, CH)],
                       hout[l].at[pl.ds(ob + k * CH, CH)], semZ)
    pltpu.async_copy(acc.at[pl.ds(zb + ZFULL * CH, ZREM)],
                     hout[l].at[pl.ds(ob + ZFULL * CH, ZREM)], semZ)
    for k in range(ZFULL):
      pltpu.make_async_copy(acc.at[pl.ds(zb + k * CH, CH)],
                            hout[l].at[pl.ds(ob + k * CH, CH)], semZ).wait()
    pltpu.make_async_copy(acc.at[pl.ds(zb + ZFULL * CH, ZREM)],
                          hout[l].at[pl.ds(ob + ZFULL * CH, ZREM)],
                          semZ).wait()

  for k in range(4):
    wt_idx(k)

  pltpu.sync_copy(fw_h, fw_v)
  pltpu.sync_copy(pidx_h.at[c, pl.ds(2 * s, 2)], eb0.at[pl.ds(0, 2)])

  hs = [x0_h, h1_h, h2_h, h3_h]
  steps = [(l, half, j) for l in range(4) for half in range(2)
           for j in range(2)]

  def _stage_ref(n):
    if n % 2 == 0:
      return rows1.at[pl.ds(0, 128)], semA
    return rows0.at[pl.ds(256, 128)], semB

  def _fire_final(n):
    l, half, j = steps[n]
    ref, sem = _stage_ref(n)
    pltpu.async_copy(hs[l].at[eb0.at[half, pl.ds(j * 128, 128)]], ref, sem)

  def _wait_final(n):
    l, half, j = steps[n]
    ref, sem = _stage_ref(n)
    pltpu.make_async_copy(hs[l].at[eb0.at[half, pl.ds(j * 128, 128)]],
                          ref, sem).wait()

  _fire_final(0)
  for n in range(16):
    _wait_final(n)
    if n + 1 < 16:
      _fire_final(n + 1)
    l, half, j = steps[n]
    fwl = fw_v[l, pl.ds(0, 16)]
    stg = rows1 if n % 2 == 0 else rows0
    srow = 0 if n % 2 == 0 else 256

    @pl.loop(0, 128)
    def _fuse(e, l=l, half=half, j=j, fwl=fwl, stg=stg, srow=srow):
      for h in (0, 16):
        v = fwl * stg[srow + e, pl.ds(h, 16)]
        if half == 0:
          tgt, row = rows0, j * 128 + e
        else:
          tgt, row = rows1, 128 + j * 128 + e
        if l == 0:
          tgt[row, pl.ds(h, 16)] = v
        else:
          tgt[row, pl.ds(h, 16)] = tgt[row, pl.ds(h, 16)] + v

  iota = lax.iota(jnp.int32, 16)

  @pl.loop(0, PB // 16)
  def _reduce(grp):
    accv = jnp.zeros((16,), jnp.float32)
    for i in range(16):
      e = grp * 16 + i
      v = (rows0[e, pl.ds(0, 16)] * rows1[128 + e, pl.ds(0, 16)]
           + rows0[e, pl.ds(16, 16)] * rows1[128 + e, pl.ds(16, 16)])
      accv = accv + jnp.where(iota == i, jnp.sum(v), 0.0)
    rows0[grp // 2, pl.ds((grp % 2) * 16, 16)] = accv

  pltpu.sync_copy(rows0.at[pl.ds(0, PB // HALF)],
                  gam_h.at[pl.ds((c * NS + s) * (PB // HALF), PB // HALF)])


@jax.jit
def _run(x0, epk, pidx, fwv):
  mesh = plsc.VectorSubcoreMesh(core_axis_name="c", subcore_axis_name="s",
                                num_cores=NC, num_subcores=NS)
  f = pl.kernel(
      _body,
      out_type=[
          jax.ShapeDtypeStruct((NC * BATCH // HALF, HALF), jnp.float32),
          jax.ShapeDtypeStruct((NC * NP, HALF), jnp.float32),
          jax.ShapeDtypeStruct((NC * NP, HALF), jnp.float32),
          jax.ShapeDtypeStruct((NC * NP, HALF), jnp.float32),
      ],
      mesh=mesh,
      compiler_params=pltpu.CompilerParams(use_tc_tiling_on_sc=False, needs_layout_passes=False),
      scratch_types=[
          pltpu.VMEM((3, CH), jnp.int32),
          pltpu.VMEM((3, CH), jnp.int32),
          pltpu.VMEM((3, CH), jnp.int32),
          pltpu.VMEM((3, CH), jnp.int32),
          pltpu.VMEM((CH, HALF), jnp.float32),
          pltpu.VMEM((CH, HALF), jnp.float32),
          pltpu.VMEM((8, 16), jnp.float32),
          pltpu.VMEM_SHARED((NP, HALF), jnp.float32),
          pltpu.SemaphoreType.DMA,
          pltpu.SemaphoreType.DMA,
          pltpu.SemaphoreType.DMA,
          pltpu.SemaphoreType.DMA,
          pltpu.SemaphoreType.DMA,
          pltpu.SemaphoreType.DMA,
          pltpu.SemaphoreType.DMA,
      ],
  )
  return f(x0, epk, pidx, fwv)


def kernel(users, items, user_emb, item_emb, edge_src, edge_dst, edge_w,
           fw1, fw2, fw3, fw4):
  all_emb = jnp.concatenate([user_emb, item_emb], axis=0)
  npad = jnp.zeros((NP - NN, HALF), jnp.float32)
  x0 = jnp.concatenate(
      [all_emb[:, :HALF], npad, all_emb[:, HALF:], npad], 0)

  pad = EPAD - E0
  esrc = jnp.concatenate([edge_src.astype(jnp.int32),
                          jnp.zeros((pad,), jnp.int32)]).reshape(
                              NS * NCHUNK, CH)
  edst = jnp.concatenate([edge_dst.astype(jnp.int32),
                          jnp.zeros((pad,), jnp.int32)]).reshape(
                              NS * NCHUNK, CH)
  wbits = lax.bitcast_convert_type(
      jnp.concatenate([edge_w.astype(jnp.float32),
                       jnp.zeros((pad,), jnp.float32)]),
      jnp.int32).reshape(NS * NCHUNK, CH)
  epk = jnp.stack([
      jnp.stack([esrc, edst, wbits], axis=1),
      jnp.stack([esrc + NP, edst, wbits], axis=1),
  ], axis=0)

  zpad = jnp.zeros((NS, CH - PB), jnp.int32)
  u2 = jnp.concatenate([users.astype(jnp.int32).reshape(NS, PB), zpad], 1)
  i2 = jnp.concatenate([(items.astype(jnp.int32) + NU).reshape(NS, PB),
                        zpad], 1)
  pidx0 = jnp.stack([u2, i2], axis=1).reshape(NS * 2, CH)
  pidx = jnp.stack([pidx0, pidx0 + NP], axis=0)

  fwv = jnp.zeros((8, 16), jnp.float32)
  fwv = fwv.at[0:4].set(
      jnp.broadcast_to(
          jnp.stack([fw1, fw2, fw3, fw4]).reshape(4, 1).astype(jnp.float32),
          (4, 16)))

  gam, _, _, _ = _run(x0, epk, pidx, fwv)
  gam = gam.reshape(NC, BATCH)
  return gam[0] + gam[1]


# speedup vs baseline: 1.3219x; 1.0638x over previous
import jax
import jax.numpy as jnp
from jax import lax
from jax.experimental import pallas as pl
from jax.experimental.pallas import tpu as pltpu
from jax.experimental.pallas import tpu_sc as plsc

NU = 25000
NN = 50000
NP = 50048
HALF = 32
E0 = 800000
BATCH = 4096
NC, NS = 2, 16
CH = 384
NCHUNK = 132
PT = NCHUNK * CH
EPAD = PT * NS
ZROWS = NP // NS
ZFULL = ZROWS // CH
ZREM = ZROWS % CH
PB = BATCH // NS


def _body(x0_h, epk_h, ew_h, pidx_h, fw_h,
          gam_h, h1_h, h2_h, h3_h,
          eb0, eb1, eb2, eb3, w0, w1, w2, w3, rows0, rows1,
          fw_v, acc, semA, semB, semI0, semI1, semI2, semI3, semZ):
  c = lax.axis_index("c")
  s = lax.axis_index("s")

  ebs = [eb0, eb1, eb2, eb3]
  ws = [w0, w1, w2, w3]
  semIs = [semI0, semI1, semI2, semI3]
  zeros16 = jnp.zeros((16,), jnp.float32)

  def pf_idx(g, k):
    gg = jnp.where(g < NCHUNK, g, 0)
    pltpu.async_copy(epk_h.at[c, s * NCHUNK + gg], ebs[k], semIs[k])
    pltpu.async_copy(ew_h.at[pl.ds(s * PT + gg * CH, CH)], ws[k], semIs[k])

  def wt_idx(k):
    pltpu.make_async_copy(epk_h.at[c, 0], ebs[k], semIs[k]).wait()
    pltpu.make_async_copy(ew_h.at[pl.ds(0, CH)], ws[k], semIs[k]).wait()

  def fire_gather(k, rowsb, sem, tab):
    pltpu.async_copy(tab.at[ebs[k].at[0]], rowsb, sem)

  def wt_gather(k, rowsb, sem, tab):
    pltpu.make_async_copy(tab.at[ebs[k].at[0]], rowsb, sem).wait()

  def scale(k, rowsb):
    wb = ws[k]

    @pl.loop(0, CH // 16)
    def _scale(gg):
      wv = wb[pl.ds(gg * 16, 16)]
      for i in range(16):
        e = gg * 16 + i
        w = wv[i]
        rowsb[e, pl.ds(0, 16)] = rowsb[e, pl.ds(0, 16)] * w
        rowsb[e, pl.ds(16, 16)] = rowsb[e, pl.ds(16, 16)] * w

  def scatter(k, rowsb):
    pltpu.sync_copy(rowsb, acc.at[ebs[k].at[1]], add=True)

  hin = [x0_h, h1_h, h2_h]
  hout = [h1_h, h2_h, h3_h]

  zb = s * ZROWS
  ob0 = zb

  for k in range(4):
    pf_idx(jnp.int32(k), k)

  for l in range(3):
    @pl.loop(0, CH)
    def _zero(e):
      rows0[e, pl.ds(0, 16)] = zeros16
      rows0[e, pl.ds(16, 16)] = zeros16

    for k in range(ZFULL):
      pltpu.async_copy(rows0, acc.at[pl.ds(zb + k * CH, CH)], semZ)
    pltpu.async_copy(rows0.at[pl.ds(0, ZREM)],
                     acc.at[pl.ds(zb + ZFULL * CH, ZREM)], semZ)
    for k in range(ZFULL):
      pltpu.make_async_copy(rows0, acc.at[pl.ds(zb + k * CH, CH)],
                            semZ).wait()
    pltpu.make_async_copy(rows0.at[pl.ds(0, ZREM)],
                          acc.at[pl.ds(zb + ZFULL * CH, ZREM)], semZ).wait()
    plsc.subcore_barrier()

    tab = hin[l]
    wt_idx(0)
    fire_gather(0, rows0, semA, tab)
    wt_idx(1)
    fire_gather(1, rows1, semB, tab)

    @pl.loop(0, NCHUNK // 4)
    def _edges(u):
      g = 4 * u
      wt_gather(0, rows0, semA, tab)
      scale(0, rows0)
      scatter(0, rows0)
      wt_idx(2)
      fire_gather(2, rows0, semA, tab)
      pf_idx(g + 4, 0)
      wt_gather(1, rows1, semB, tab)
      scale(1, rows1)
      scatter(1, rows1)
      wt_idx(3)
      fire_gather(3, rows1, semB, tab)
      pf_idx(g + 5, 1)
      wt_gather(2, rows0, semA, tab)
      scale(2, rows0)
      scatter(2, rows0)
      wt_idx(0)
      fire_gather(0, rows0, semA, tab)
      pf_idx(g + 6, 2)
      wt_gather(3, rows1, semB, tab)
      scale(3, rows1)
      scatter(3, rows1)
      wt_idx(1)
      fire_gather(1, rows1, semB, tab)
      pf_idx(g + 7, 3)

    wt_gather(0, rows0, semA, tab)
    wt_gather(1, rows1, semB, tab)
    wt_idx(2)
    wt_idx(3)
    plsc.subcore_barrier()

    for k in range(4):
      pf_idx(jnp.int32(k), k)

    ob = c * NP + zb
    for k in range(ZFULL):
      pltpu.async_copy(acc.at[pl.ds(zb + k * CH, CH)],
                       hout[l].at[pl.ds(ob + k * CH, CH)], semZ)
    pltpu.async_copy(acc.at[pl.ds(zb + ZFULL * CH, ZREM)],
                     hout[l].at[pl.ds(ob + ZFULL * CH, ZREM)], semZ)
    for k in range(ZFULL):
      pltpu.make_async_copy(acc.at[pl.ds(zb + k * CH, CH)],
                            hout[l].at[pl.ds(ob + k * CH, CH)], semZ).wait()
    pltpu.make_async_copy(acc.at[pl.ds(zb + ZFULL * CH, ZREM)],
                          hout[l].at[pl.ds(ob + ZFULL * CH, ZREM)],
                          semZ).wait()

  for k in range(4):
    wt_idx(k)

  pltpu.sync_copy(fw_h, fw_v)
  pltpu.sync_copy(pidx_h.at[c, pl.ds(2 * s, 2)], eb0)

  hs = [x0_h, h1_h, h2_h, h3_h]
  steps = [(l, half, j) for l in range(4) for half in range(2)
           for j in range(2)]

  def _stage_ref(n):
    if n % 2 == 0:
      return rows1.at[pl.ds(0, 128)], semA
    return rows0.at[pl.ds(256, 128)], semB

  def _fire_final(n):
    l, half, j = steps[n]
    ref, sem = _stage_ref(n)
    pltpu.async_copy(hs[l].at[eb0.at[half, pl.ds(j * 128, 128)]], ref, sem)

  def _wait_final(n):
    l, half, j = steps[n]
    ref, sem = _stage_ref(n)
    pltpu.make_async_copy(hs[l].at[eb0.at[half, pl.ds(j * 128, 128)]],
                          ref, sem).wait()

  _fire_final(0)
  for n in range(16):
    _wait_final(n)
    if n + 1 < 16:
      _fire_final(n + 1)
    l, half, j = steps[n]
    fwl = fw_v[l, pl.ds(0, 16)]
    stg = rows1 if n % 2 == 0 else rows0
    srow = 0 if n % 2 == 0 else 256

    @pl.loop(0, 128)
    def _fuse(e, l=l, half=half, j=j, fwl=fwl, stg=stg, srow=srow):
      for h in (0, 16):
        v = fwl * stg[srow + e, pl.ds(h, 16)]
        if half == 0:
          tgt, row = rows0, j * 128 + e
        else:
          tgt, row = rows1, 128 + j * 128 + e
        if l == 0:
          tgt[row, pl.ds(h, 16)] = v
        else:
          tgt[row, pl.ds(h, 16)] = tgt[row, pl.ds(h, 16)] + v

  iota = lax.iota(jnp.int32, 16)

  @pl.loop(0, PB // 16)
  def _reduce(grp):
    accv = jnp.zeros((16,), jnp.float32)
    for i in range(16):
      e = grp * 16 + i
      v = (rows0[e, pl.ds(0, 16)] * rows1[128 + e, pl.ds(0, 16)]
           + rows0[e, pl.ds(16, 16)] * rows1[128 + e, pl.ds(16, 16)])
      sv = v[0]
      for q in range(1, 16):
        sv = sv + v[q]
      accv = accv + jnp.where(iota == i, sv, 0.0)
    rows0[grp // 2, pl.ds((grp % 2) * 16, 16)] = accv

  pltpu.sync_copy(rows0.at[pl.ds(0, PB // HALF)],
                  gam_h.at[pl.ds((c * NS + s) * (PB // HALF), PB // HALF)])


@jax.jit
def _run(x0, epk, ew, pidx, fwv):
  mesh = plsc.VectorSubcoreMesh(core_axis_name="c", subcore_axis_name="s",
                                num_cores=NC, num_subcores=NS)
  f = pl.kernel(
      _body,
      out_type=[
          jax.ShapeDtypeStruct((NC * BATCH // HALF, HALF), jnp.float32),
          jax.ShapeDtypeStruct((NC * NP, HALF), jnp.float32),
          jax.ShapeDtypeStruct((NC * NP, HALF), jnp.float32),
          jax.ShapeDtypeStruct((NC * NP, HALF), jnp.float32),
      ],
      mesh=mesh,
      compiler_params=pltpu.CompilerParams(use_tc_tiling_on_sc=False),
      scratch_types=[
          pltpu.VMEM((2, CH), jnp.int32),
          pltpu.VMEM((2, CH), jnp.int32),
          pltpu.VMEM((2, CH), jnp.int32),
          pltpu.VMEM((2, CH), jnp.int32),
          pltpu.VMEM((CH,), jnp.float32),
          pltpu.VMEM((CH,), jnp.float32),
          pltpu.VMEM((CH,), jnp.float32),
          pltpu.VMEM((CH,), jnp.float32),
          pltpu.VMEM((CH, HALF), jnp.float32),
          pltpu.VMEM((CH, HALF), jnp.float32),
          pltpu.VMEM((8, 16), jnp.float32),
          pltpu.VMEM_SHARED((NP, HALF), jnp.float32),
          pltpu.SemaphoreType.DMA,
          pltpu.SemaphoreType.DMA,
          pltpu.SemaphoreType.DMA,
          pltpu.SemaphoreType.DMA,
          pltpu.SemaphoreType.DMA,
          pltpu.SemaphoreType.DMA,
          pltpu.SemaphoreType.DMA,
      ],
  )
  return f(x0, epk, ew, pidx, fwv)


def kernel(users, items, user_emb, item_emb, edge_src, edge_dst, edge_w,
           fw1, fw2, fw3, fw4):
  all_emb = jnp.concatenate([user_emb, item_emb], axis=0)
  npad = jnp.zeros((NP - NN, HALF), jnp.float32)
  x0 = jnp.concatenate(
      [all_emb[:, :HALF], npad, all_emb[:, HALF:], npad], 0)

  pad = EPAD - E0
  esrc = jnp.concatenate([edge_src.astype(jnp.int32),
                          jnp.zeros((pad,), jnp.int32)]).reshape(
                              NS * NCHUNK, CH)
  edst = jnp.concatenate([edge_dst.astype(jnp.int32),
                          jnp.zeros((pad,), jnp.int32)]).reshape(
                              NS * NCHUNK, CH)
  ew = jnp.concatenate([edge_w.astype(jnp.float32),
                        jnp.zeros((pad,), jnp.float32)])
  epk = jnp.stack([
      jnp.stack([esrc, edst], axis=1),
      jnp.stack([esrc + NP, edst], axis=1),
  ], axis=0)

  zpad = jnp.zeros((NS, CH - PB), jnp.int32)
  u2 = jnp.concatenate([users.astype(jnp.int32).reshape(NS, PB), zpad], 1)
  i2 = jnp.concatenate([(items.astype(jnp.int32) + NU).reshape(NS, PB),
                        zpad], 1)
  pidx0 = jnp.stack([u2, i2], axis=1).reshape(NS * 2, CH)
  pidx = jnp.stack([pidx0, pidx0 + NP], axis=0)

  fwv = jnp.zeros((8, 16), jnp.float32)
  fwv = fwv.at[0:4].set(
      jnp.broadcast_to(
          jnp.stack([fw1, fw2, fw3, fw4]).reshape(4, 1).astype(jnp.float32),
          (4, 16)))

  gam, _, _, _ = _run(x0, epk, ew, pidx, fwv)
  gam = gam.reshape(NC, BATCH)
  return gam[0] + gam[1]
